# Initial kernel scaffold; baseline (speedup 1.0000x reference)
import jax
import jax.numpy as jnp
from jax.experimental import pallas as pl


def kernel(x, edge_index, edge_weight, h_0, c_0, ggc_w, gru_wi, gru_wh, gru_bi, gru_bh, lstm_wi, lstm_wh, lstm_bi, lstm_bh, lin_w, lin_b):
    raise NotImplementedError("write your pallas kernel here")


# speedup vs baseline: 21.5427x; 21.5427x over previous
import functools

import jax
import jax.numpy as jnp
from jax import lax
from jax.experimental import pallas as pl
from jax.experimental.pallas import tpu as pltpu
from jax.experimental.pallas import tpu_sc as plsc

N = 100000
E = 6400000
EPAD = 6553600
GROUP = 128
GROUPS_PER_CHUNK = 16
CHUNK = GROUP * GROUPS_PER_CHUNK
CHUNKS_PER_TILE = EPAD // (32 * CHUNK)
ROWS2D = EPAD // GROUP
R = 100352
RSLICE = R // 16


def _sc_body(xtab, src2d, dst2d, wflat, zeros, out, table, vsrc, vdst, vw, rows,
             gsem, ssem):
    cid = lax.axis_index("c")
    sid = lax.axis_index("s")
    wid = cid * 16 + sid

    @pl.when(sid == 0)
    def _zero():
        pltpu.sync_copy(zeros, table)

    plsc.subcore_barrier()

    iota = lax.broadcasted_iota(jnp.int32, (16,), 0)
    ci = iota & 7
    base_r = iota >> 3
    m_lo = ci < 4
    c4 = jnp.where(ci == 4, 1.0, 0.0).astype(jnp.float32)

    def chunk_body(c, carry):
        row0 = wid * (CHUNKS_PER_TILE * GROUPS_PER_CHUNK) + c * GROUPS_PER_CHUNK
        e0 = wid * (CHUNKS_PER_TILE * CHUNK) + c * CHUNK
        pltpu.sync_copy(src2d.at[pl.ds(row0, GROUPS_PER_CHUNK)], vsrc)
        pltpu.sync_copy(dst2d.at[pl.ds(row0, GROUPS_PER_CHUNK)], vdst)
        pltpu.sync_copy(wflat.at[pl.ds(e0, CHUNK)], vw)

        descs = []
        for g in range(GROUPS_PER_CHUNK):
            descs.append(pltpu.async_copy(xtab.at[vsrc.at[g]], rows.at[g], gsem))
        for d in descs:
            d.wait()

        def mul_body(t, _):
            g_s = jnp.full((16,), t >> 6, jnp.int32)
            ri = (t & 63) * 2 + base_r
            widx = 2 * t + base_r
            wb = plsc.load_gather(vw, [widx])
            v = plsc.load_gather(rows, [g_s, ri, ci])
            v2 = jnp.where(m_lo, v * wb, c4)
            plsc.store_scatter(rows, [g_s, ri, ci], v2)
            return 0

        lax.fori_loop(0, CHUNK // 2, mul_body, 0)

        descs = []
        for g in range(GROUPS_PER_CHUNK):
            descs.append(
                pltpu.async_copy(rows.at[g], table.at[vdst.at[g]], ssem, add=True))
        for d in descs:
            d.wait()
        return carry

    lax.fori_loop(0, CHUNKS_PER_TILE, chunk_body, 0)

    plsc.subcore_barrier()
    pltpu.sync_copy(table.at[pl.ds(sid * RSLICE, RSLICE)],
                    out.at[cid, pl.ds(sid * RSLICE, RSLICE)])


def _sc_aggregate(xtab, src2d, dst2d, wflat, zeros):
    mesh = plsc.VectorSubcoreMesh(core_axis_name="c", subcore_axis_name="s")
    return pl.kernel(
        _sc_body,
        out_type=jax.ShapeDtypeStruct((2, R, 8), jnp.float32),
        mesh=mesh,
        compiler_params=pltpu.CompilerParams(needs_layout_passes=False,
                                             use_tc_tiling_on_sc=False),
        scratch_types=[
            pltpu.VMEM_SHARED((R, 8), jnp.float32),
            pltpu.VMEM((GROUPS_PER_CHUNK, GROUP), jnp.int32),
            pltpu.VMEM((GROUPS_PER_CHUNK, GROUP), jnp.int32),
            pltpu.VMEM((CHUNK,), jnp.float32),
            pltpu.VMEM((GROUPS_PER_CHUNK, GROUP, 8), jnp.float32),
            pltpu.SemaphoreType.DMA,
            pltpu.SemaphoreType.DMA,
        ],
    )(xtab, src2d, dst2d, wflat, zeros)


BLK = 4000


def _tc_body(x_r, p_r, h0_r, c0_r, ggc_r, wi_r, wh_r, bi_r, bh_r,
             lwi_r, lwh_r, lb_r, lin_r, linb_r, out_r, ht_r, ct_r):
    p = p_r[...]
    aggx = p[0, :, 0:4] + p[1, :, 0:4]
    cnt = p[0, :, 4:5] + p[1, :, 4:5]
    agg = jnp.dot(aggx, ggc_r[...], preferred_element_type=jnp.float32)
    agg = agg / jnp.maximum(cnt, 1.0)
    xb = x_r[...]
    gi = jnp.dot(agg, wi_r[...], preferred_element_type=jnp.float32) + bi_r[...]
    gh = jnp.dot(xb, wh_r[...], preferred_element_type=jnp.float32) + bh_r[...]
    r = jax.nn.sigmoid(gi[:, 0:4] + gh[:, 0:4])
    z = jax.nn.sigmoid(gi[:, 4:8] + gh[:, 4:8])
    nc = jnp.tanh(gi[:, 8:12] + r * gh[:, 8:12])
    hc = (1.0 - z) * nc + z * xb
    gates = (jnp.dot(hc, lwi_r[...], preferred_element_type=jnp.float32)
             + jnp.dot(h0_r[...], lwh_r[...], preferred_element_type=jnp.float32)
             + lb_r[...])
    i_t = jax.nn.sigmoid(gates[:, 0:32])
    f_t = jax.nn.sigmoid(gates[:, 32:64])
    g_t = jnp.tanh(gates[:, 64:96])
    o_t = jax.nn.sigmoid(gates[:, 96:128])
    c_t = f_t * c0_r[...] + i_t * g_t
    h_t = o_t * jnp.tanh(c_t)
    ct_r[...] = c_t
    ht_r[...] = h_t
    out_r[...] = (jnp.sum(jax.nn.relu(h_t) * lin_r[...], axis=1, keepdims=True)
                  + linb_r[...])


def _tc_dense(x, part, h_0, c_0, ggc_w, gru_wiT, gru_whT, gru_bi2, gru_bh2,
              lstm_wiT, lstm_whT, lstm_b2, lin_w, lin_b2):
    nblk = N // BLK
    full = lambda a: pl.BlockSpec(a.shape, lambda i: (0,) * a.ndim)
    return pl.pallas_call(
        _tc_body,
        grid=(nblk,),
        in_specs=[
            pl.BlockSpec((BLK, 4), lambda i: (i, 0)),
            pl.BlockSpec((2, BLK, 8), lambda i: (0, i, 0)),
            pl.BlockSpec((BLK, 32), lambda i: (i, 0)),
            pl.BlockSpec((BLK, 32), lambda i: (i, 0)),
            full(ggc_w), full(gru_wiT), full(gru_whT), full(gru_bi2),
            full(gru_bh2), full(lstm_wiT), full(lstm_whT), full(lstm_b2),
            full(lin_w), full(lin_b2),
        ],
        out_specs=[
            pl.BlockSpec((BLK, 1), lambda i: (i, 0)),
            pl.BlockSpec((BLK, 32), lambda i: (i, 0)),
            pl.BlockSpec((BLK, 32), lambda i: (i, 0)),
        ],
        out_shape=[
            jax.ShapeDtypeStruct((N, 1), jnp.float32),
            jax.ShapeDtypeStruct((N, 32), jnp.float32),
            jax.ShapeDtypeStruct((N, 32), jnp.float32),
        ],
    )(x, part, h_0, c_0, ggc_w, gru_wiT, gru_whT, gru_bi2, gru_bh2,
      lstm_wiT, lstm_whT, lstm_b2, lin_w, lin_b2)


def kernel(x, edge_index, edge_weight, h_0, c_0, ggc_w, gru_wi, gru_wh,
           gru_bi, gru_bh, lstm_wi, lstm_wh, lstm_bi, lstm_bh, lin_w, lin_b):
    pad = EPAD - E
    src_p = jnp.concatenate([edge_index[0], jnp.zeros((pad,), jnp.int32)])
    dst_p = jnp.concatenate([edge_index[1], jnp.full((pad,), N, jnp.int32)])
    w_p = jnp.concatenate([edge_weight, jnp.zeros((pad,), jnp.float32)])
    src2d = src_p.reshape(ROWS2D, GROUP)
    dst2d = dst_p.reshape(ROWS2D, GROUP)
    xtab = jnp.concatenate([x, jnp.zeros((N, 4), jnp.float32)], axis=1)
    zeros = jnp.zeros((R, 8), jnp.float32)

    part = _sc_aggregate(xtab, src2d, dst2d, w_p, zeros)

    out, h_t, c_t = _tc_dense(
        x, part, h_0, c_0, ggc_w,
        gru_wi.T, gru_wh.T, gru_bi.reshape(1, 12), gru_bh.reshape(1, 12),
        lstm_wi.T, lstm_wh.T, (lstm_bi + lstm_bh).reshape(1, 128),
        lin_w, lin_b.reshape(1, 1))
    return (out, h_t, c_t)


# speedup vs baseline: 23.9967x; 1.1139x over previous
import functools

import jax
import jax.numpy as jnp
from jax import lax
from jax.experimental import pallas as pl
from jax.experimental.pallas import tpu as pltpu
from jax.experimental.pallas import tpu_sc as plsc

N = 100000
E = 6400000
EPAD = 6553600
GROUP = 128
GROUPS_PER_CHUNK = 8
CHUNK = GROUP * GROUPS_PER_CHUNK
CHUNKS_PER_TILE = EPAD // (32 * CHUNK)
ROWS2D = EPAD // GROUP
R = 100352
RSLICE = R // 16


NBUF = 4


def _sc_body(xtab, src2d, dst2d, wflat, zeros, out, table, vsrc, vdst, vw, rows,
             gsem, ssem, isem):
    cid = lax.axis_index("c")
    sid = lax.axis_index("s")
    wid = cid * 16 + sid

    @pl.when(sid == 0)
    def _zero():
        pltpu.sync_copy(zeros, table)

    plsc.subcore_barrier()

    iota = lax.broadcasted_iota(jnp.int32, (16,), 0)
    ci = iota & 7
    base_r = iota >> 3
    m_lo = ci < 4
    c4 = jnp.where(ci == 4, 1.0, 0.0).astype(jnp.float32)

    def fire_idx(c, b):
        row0 = wid * (CHUNKS_PER_TILE * GROUPS_PER_CHUNK) + c * GROUPS_PER_CHUNK
        e0 = wid * (CHUNKS_PER_TILE * CHUNK) + c * CHUNK
        pltpu.async_copy(src2d.at[pl.ds(row0, GROUPS_PER_CHUNK)], vsrc.at[b], isem)
        pltpu.async_copy(dst2d.at[pl.ds(row0, GROUPS_PER_CHUNK)], vdst.at[b], isem)
        pltpu.async_copy(wflat.at[pl.ds(e0, CHUNK)], vw.at[b], isem)

    def wait_idx(b):
        for _ in range(3):
            pltpu.make_async_copy(src2d.at[pl.ds(0, GROUPS_PER_CHUNK)],
                                  vsrc.at[b], isem).wait()

    def fire_gathers(b):
        for g in range(GROUPS_PER_CHUNK):
            pltpu.async_copy(xtab.at[vsrc.at[b, g]], rows.at[b, g], gsem)

    def wait_gathers(b):
        for g in range(GROUPS_PER_CHUNK):
            pltpu.make_async_copy(xtab.at[vsrc.at[b, g]], rows.at[b, g],
                                  gsem).wait()

    def fire_scatters(b):
        for g in range(GROUPS_PER_CHUNK):
            pltpu.async_copy(rows.at[b, g], table.at[vdst.at[b, g]], ssem,
                             add=True)

    def wait_scatters(b):
        for g in range(GROUPS_PER_CHUNK):
            pltpu.make_async_copy(rows.at[b, g], table.at[vdst.at[b, g]],
                                  ssem).wait()

    def mul(b):
        b_s = jnp.full((16,), b, jnp.int32)
        for g in range(GROUPS_PER_CHUNK):
            g_s = jnp.full((16,), g, jnp.int32)
            wbase = g * GROUP + base_r

            def _mb(i, carry):
                ri = 2 * i + base_r
                wb = plsc.load_gather(vw, [b_s, wbase + 2 * i])
                v = plsc.load_gather(rows, [b_s, g_s, ri, ci])
                v2 = jnp.where(m_lo, v * wb, c4)
                plsc.store_scatter(rows, [b_s, g_s, ri, ci], v2)
                return carry

            lax.fori_loop(0, GROUP // 2, _mb, 0)

    fire_idx(0, 0)
    wait_idx(0)
    fire_gathers(0)
    fire_idx(1, 1)

    def super_body(s, carry):
        for b in range(NBUF):
            c = NBUF * s + b
            b1 = (b + 1) % NBUF
            b2 = (b + 2) % NBUF

            @pl.when(c >= 2)
            def _():
                wait_scatters(b2)

            @pl.when(c + 2 < CHUNKS_PER_TILE)
            def _():
                fire_idx(c + 2, b2)

            wait_gathers(b)
            mul(b)

            @pl.when(c + 1 < CHUNKS_PER_TILE)
            def _():
                wait_idx(b1)
                fire_gathers(b1)

            fire_scatters(b)
        return carry

    lax.fori_loop(0, CHUNKS_PER_TILE // NBUF, super_body, 0)
    wait_scatters((CHUNKS_PER_TILE - 2) % NBUF)
    wait_scatters((CHUNKS_PER_TILE - 1) % NBUF)

    plsc.subcore_barrier()
    pltpu.sync_copy(table.at[pl.ds(sid * RSLICE, RSLICE)],
                    out.at[cid, pl.ds(sid * RSLICE, RSLICE)])


def _sc_aggregate(xtab, src2d, dst2d, wflat, zeros):
    mesh = plsc.VectorSubcoreMesh(core_axis_name="c", subcore_axis_name="s")
    return pl.kernel(
        _sc_body,
        out_type=jax.ShapeDtypeStruct((2, R, 8), jnp.float32),
        mesh=mesh,
        compiler_params=pltpu.CompilerParams(needs_layout_passes=False,
                                             use_tc_tiling_on_sc=False),
        scratch_types=[
            pltpu.VMEM_SHARED((R, 8), jnp.float32),
            pltpu.VMEM((NBUF, GROUPS_PER_CHUNK, GROUP), jnp.int32),
            pltpu.VMEM((NBUF, GROUPS_PER_CHUNK, GROUP), jnp.int32),
            pltpu.VMEM((NBUF, CHUNK), jnp.float32),
            pltpu.VMEM((NBUF, GROUPS_PER_CHUNK, GROUP, 8), jnp.float32),
            pltpu.SemaphoreType.DMA,
            pltpu.SemaphoreType.DMA,
            pltpu.SemaphoreType.DMA,
        ],
    )(xtab, src2d, dst2d, wflat, zeros)


BLK = 4000


def _tc_body(x_r, p_r, h0_r, c0_r, ggc_r, wi_r, wh_r, bi_r, bh_r,
             lwi_r, lwh_r, lb_r, lin_r, linb_r, out_r, ht_r, ct_r):
    p = p_r[...]
    aggx = p[0, :, 0:4] + p[1, :, 0:4]
    cnt = p[0, :, 4:5] + p[1, :, 4:5]
    agg = jnp.dot(aggx, ggc_r[...], preferred_element_type=jnp.float32)
    agg = agg / jnp.maximum(cnt, 1.0)
    xb = x_r[...]
    gi = jnp.dot(agg, wi_r[...], preferred_element_type=jnp.float32) + bi_r[...]
    gh = jnp.dot(xb, wh_r[...], preferred_element_type=jnp.float32) + bh_r[...]
    r = jax.nn.sigmoid(gi[:, 0:4] + gh[:, 0:4])
    z = jax.nn.sigmoid(gi[:, 4:8] + gh[:, 4:8])
    nc = jnp.tanh(gi[:, 8:12] + r * gh[:, 8:12])
    hc = (1.0 - z) * nc + z * xb
    gates = (jnp.dot(hc, lwi_r[...], preferred_element_type=jnp.float32)
             + jnp.dot(h0_r[...], lwh_r[...], preferred_element_type=jnp.float32)
             + lb_r[...])
    i_t = jax.nn.sigmoid(gates[:, 0:32])
    f_t = jax.nn.sigmoid(gates[:, 32:64])
    g_t = jnp.tanh(gates[:, 64:96])
    o_t = jax.nn.sigmoid(gates[:, 96:128])
    c_t = f_t * c0_r[...] + i_t * g_t
    h_t = o_t * jnp.tanh(c_t)
    ct_r[...] = c_t
    ht_r[...] = h_t
    out_r[...] = (jnp.sum(jax.nn.relu(h_t) * lin_r[...], axis=1, keepdims=True)
                  + linb_r[...])


def _tc_dense(x, part, h_0, c_0, ggc_w, gru_wiT, gru_whT, gru_bi2, gru_bh2,
              lstm_wiT, lstm_whT, lstm_b2, lin_w, lin_b2):
    nblk = N // BLK
    full = lambda a: pl.BlockSpec(a.shape, lambda i: (0,) * a.ndim)
    return pl.pallas_call(
        _tc_body,
        grid=(nblk,),
        in_specs=[
            pl.BlockSpec((BLK, 4), lambda i: (i, 0)),
            pl.BlockSpec((2, BLK, 8), lambda i: (0, i, 0)),
            pl.BlockSpec((BLK, 32), lambda i: (i, 0)),
            pl.BlockSpec((BLK, 32), lambda i: (i, 0)),
            full(ggc_w), full(gru_wiT), full(gru_whT), full(gru_bi2),
            full(gru_bh2), full(lstm_wiT), full(lstm_whT), full(lstm_b2),
            full(lin_w), full(lin_b2),
        ],
        out_specs=[
            pl.BlockSpec((BLK, 1), lambda i: (i, 0)),
            pl.BlockSpec((BLK, 32), lambda i: (i, 0)),
            pl.BlockSpec((BLK, 32), lambda i: (i, 0)),
        ],
        out_shape=[
            jax.ShapeDtypeStruct((N, 1), jnp.float32),
            jax.ShapeDtypeStruct((N, 32), jnp.float32),
            jax.ShapeDtypeStruct((N, 32), jnp.float32),
        ],
    )(x, part, h_0, c_0, ggc_w, gru_wiT, gru_whT, gru_bi2, gru_bh2,
      lstm_wiT, lstm_whT, lstm_b2, lin_w, lin_b2)


def kernel(x, edge_index, edge_weight, h_0, c_0, ggc_w, gru_wi, gru_wh,
           gru_bi, gru_bh, lstm_wi, lstm_wh, lstm_bi, lstm_bh, lin_w, lin_b):
    pad = EPAD - E
    src_p = jnp.concatenate([edge_index[0], jnp.zeros((pad,), jnp.int32)])
    dst_p = jnp.concatenate([edge_index[1], jnp.full((pad,), N, jnp.int32)])
    w_p = jnp.concatenate([edge_weight, jnp.zeros((pad,), jnp.float32)])
    src2d = src_p.reshape(ROWS2D, GROUP)
    dst2d = dst_p.reshape(ROWS2D, GROUP)
    xtab = jnp.concatenate([x, jnp.zeros((N, 4), jnp.float32)], axis=1)
    zeros = jnp.zeros((R, 8), jnp.float32)

    part = _sc_aggregate(xtab, src2d, dst2d, w_p, zeros)

    out, h_t, c_t = _tc_dense(
        x, part, h_0, c_0, ggc_w,
        gru_wi.T, gru_wh.T, gru_bi.reshape(1, 12), gru_bh.reshape(1, 12),
        lstm_wi.T, lstm_wh.T, (lstm_bi + lstm_bh).reshape(1, 128),
        lin_w, lin_b.reshape(1, 1))
    return (out, h_t, c_t)


# speedup vs baseline: 27.4352x; 1.1433x over previous
import functools

import jax
import jax.numpy as jnp
from jax import lax
from jax.experimental import pallas as pl
from jax.experimental.pallas import tpu as pltpu
from jax.experimental.pallas import tpu_sc as plsc

N = 100000
E = 6400000
EPAD = 6553600
GROUP = 128
GROUPS_PER_CHUNK = 8
CHUNK = GROUP * GROUPS_PER_CHUNK
CHUNKS_PER_TILE = EPAD // (32 * CHUNK)
ROWS2D = EPAD // GROUP
R = 100352
RSLICE = R // 16


NBUF = 4


def _sc_body(xtab, src2d, dst2d, wflat, zeros, out, table, vsrc, vdst, vw, rows,
             gsem, ssem, isem):
    cid = lax.axis_index("c")
    sid = lax.axis_index("s")
    wid = cid * 16 + sid

    @pl.when(sid == 0)
    def _zero():
        pltpu.sync_copy(zeros, table)

    plsc.subcore_barrier()

    iota = lax.broadcasted_iota(jnp.int32, (16,), 0)
    ci4 = iota & 3
    r4 = iota >> 2

    def fire_idx(c, b):
        row0 = wid * (CHUNKS_PER_TILE * GROUPS_PER_CHUNK) + c * GROUPS_PER_CHUNK
        e0 = wid * (CHUNKS_PER_TILE * CHUNK) + c * CHUNK
        pltpu.async_copy(src2d.at[pl.ds(row0, GROUPS_PER_CHUNK)], vsrc.at[b], isem)
        pltpu.async_copy(dst2d.at[pl.ds(row0, GROUPS_PER_CHUNK)], vdst.at[b], isem)
        pltpu.async_copy(wflat.at[pl.ds(e0, CHUNK)], vw.at[b], isem)

    def wait_idx(b):
        for _ in range(3):
            pltpu.make_async_copy(src2d.at[pl.ds(0, GROUPS_PER_CHUNK)],
                                  vsrc.at[b], isem).wait()

    def fire_gathers(b):
        for g in range(GROUPS_PER_CHUNK):
            pltpu.async_copy(xtab.at[vsrc.at[b, g]], rows.at[b, g], gsem)

    def wait_gathers(b):
        for g in range(GROUPS_PER_CHUNK):
            pltpu.make_async_copy(xtab.at[vsrc.at[b, g]], rows.at[b, g],
                                  gsem).wait()

    def fire_scatters(b):
        for g in range(GROUPS_PER_CHUNK):
            pltpu.async_copy(rows.at[b, g], table.at[vdst.at[b, g]], ssem,
                             add=True)

    def wait_scatters(b):
        for g in range(GROUPS_PER_CHUNK):
            pltpu.make_async_copy(rows.at[b, g], table.at[vdst.at[b, g]],
                                  ssem).wait()

    def mul(b):
        b_s = jnp.full((16,), b, jnp.int32)
        for g in range(GROUPS_PER_CHUNK):
            g_s = jnp.full((16,), g, jnp.int32)

            def _mb(i, carry):
                s = 32 * i
                for u in range(8):
                    ri = (s + 4 * u) + r4
                    wb = plsc.load_gather(vw, [b_s, g * GROUP + ri])
                    v = plsc.load_gather(rows, [b_s, g_s, ri, ci4])
                    plsc.store_scatter(rows, [b_s, g_s, ri, ci4], v * wb)
                return carry

            lax.fori_loop(0, GROUP // 32, _mb, 0)

    fire_idx(0, 0)
    wait_idx(0)
    fire_gathers(0)
    fire_idx(1, 1)

    def super_body(s, carry):
        for b in range(NBUF):
            c = NBUF * s + b
            b1 = (b + 1) % NBUF
            b2 = (b + 2) % NBUF

            @pl.when(c >= 2)
            def _():
                wait_scatters(b2)

            @pl.when(c + 2 < CHUNKS_PER_TILE)
            def _():
                fire_idx(c + 2, b2)

            wait_gathers(b)
            mul(b)

            @pl.when(c + 1 < CHUNKS_PER_TILE)
            def _():
                wait_idx(b1)
                fire_gathers(b1)

            fire_scatters(b)
        return carry

    lax.fori_loop(0, CHUNKS_PER_TILE // NBUF, super_body, 0)
    wait_scatters((CHUNKS_PER_TILE - 2) % NBUF)
    wait_scatters((CHUNKS_PER_TILE - 1) % NBUF)

    plsc.subcore_barrier()
    pltpu.sync_copy(table.at[pl.ds(sid * RSLICE, RSLICE)],
                    out.at[cid, pl.ds(sid * RSLICE, RSLICE)])


def _sc_aggregate(xtab, src2d, dst2d, wflat, zeros):
    mesh = plsc.VectorSubcoreMesh(core_axis_name="c", subcore_axis_name="s")
    return pl.kernel(
        _sc_body,
        out_type=jax.ShapeDtypeStruct((2, R, 8), jnp.float32),
        mesh=mesh,
        compiler_params=pltpu.CompilerParams(needs_layout_passes=False,
                                             use_tc_tiling_on_sc=False),
        scratch_types=[
            pltpu.VMEM_SHARED((R, 8), jnp.float32),
            pltpu.VMEM((NBUF, GROUPS_PER_CHUNK, GROUP), jnp.int32),
            pltpu.VMEM((NBUF, GROUPS_PER_CHUNK, GROUP), jnp.int32),
            pltpu.VMEM((NBUF, CHUNK), jnp.float32),
            pltpu.VMEM((NBUF, GROUPS_PER_CHUNK, GROUP, 8), jnp.float32),
            pltpu.SemaphoreType.DMA,
            pltpu.SemaphoreType.DMA,
            pltpu.SemaphoreType.DMA,
        ],
    )(xtab, src2d, dst2d, wflat, zeros)


BLK = 4000


def _tc_body(x_r, p_r, h0_r, c0_r, ggc_r, wi_r, wh_r, bi_r, bh_r,
             lwi_r, lwh_r, lb_r, lin_r, linb_r, out_r, ht_r, ct_r):
    p = p_r[...]
    aggx = p[0, :, 0:4] + p[1, :, 0:4]
    cnt = p[0, :, 4:5] + p[1, :, 4:5]
    agg = jnp.dot(aggx, ggc_r[...], preferred_element_type=jnp.float32)
    agg = agg / jnp.maximum(cnt, 1.0)
    xb = x_r[...]
    gi = jnp.dot(agg, wi_r[...], preferred_element_type=jnp.float32) + bi_r[...]
    gh = jnp.dot(xb, wh_r[...], preferred_element_type=jnp.float32) + bh_r[...]
    r = jax.nn.sigmoid(gi[:, 0:4] + gh[:, 0:4])
    z = jax.nn.sigmoid(gi[:, 4:8] + gh[:, 4:8])
    nc = jnp.tanh(gi[:, 8:12] + r * gh[:, 8:12])
    hc = (1.0 - z) * nc + z * xb
    gates = (jnp.dot(hc, lwi_r[...], preferred_element_type=jnp.float32)
             + jnp.dot(h0_r[...], lwh_r[...], preferred_element_type=jnp.float32)
             + lb_r[...])
    i_t = jax.nn.sigmoid(gates[:, 0:32])
    f_t = jax.nn.sigmoid(gates[:, 32:64])
    g_t = jnp.tanh(gates[:, 64:96])
    o_t = jax.nn.sigmoid(gates[:, 96:128])
    c_t = f_t * c0_r[...] + i_t * g_t
    h_t = o_t * jnp.tanh(c_t)
    ct_r[...] = c_t
    ht_r[...] = h_t
    out_r[...] = (jnp.sum(jax.nn.relu(h_t) * lin_r[...], axis=1, keepdims=True)
                  + linb_r[...])


def _tc_dense(x, part, h_0, c_0, ggc_w, gru_wiT, gru_whT, gru_bi2, gru_bh2,
              lstm_wiT, lstm_whT, lstm_b2, lin_w, lin_b2):
    nblk = N // BLK
    full = lambda a: pl.BlockSpec(a.shape, lambda i: (0,) * a.ndim)
    return pl.pallas_call(
        _tc_body,
        grid=(nblk,),
        in_specs=[
            pl.BlockSpec((BLK, 4), lambda i: (i, 0)),
            pl.BlockSpec((2, BLK, 8), lambda i: (0, i, 0)),
            pl.BlockSpec((BLK, 32), lambda i: (i, 0)),
            pl.BlockSpec((BLK, 32), lambda i: (i, 0)),
            full(ggc_w), full(gru_wiT), full(gru_whT), full(gru_bi2),
            full(gru_bh2), full(lstm_wiT), full(lstm_whT), full(lstm_b2),
            full(lin_w), full(lin_b2),
        ],
        out_specs=[
            pl.BlockSpec((BLK, 1), lambda i: (i, 0)),
            pl.BlockSpec((BLK, 32), lambda i: (i, 0)),
            pl.BlockSpec((BLK, 32), lambda i: (i, 0)),
        ],
        out_shape=[
            jax.ShapeDtypeStruct((N, 1), jnp.float32),
            jax.ShapeDtypeStruct((N, 32), jnp.float32),
            jax.ShapeDtypeStruct((N, 32), jnp.float32),
        ],
    )(x, part, h_0, c_0, ggc_w, gru_wiT, gru_whT, gru_bi2, gru_bh2,
      lstm_wiT, lstm_whT, lstm_b2, lin_w, lin_b2)


def kernel(x, edge_index, edge_weight, h_0, c_0, ggc_w, gru_wi, gru_wh,
           gru_bi, gru_bh, lstm_wi, lstm_wh, lstm_bi, lstm_bh, lin_w, lin_b):
    pad = EPAD - E
    src_p = jnp.concatenate([edge_index[0], jnp.zeros((pad,), jnp.int32)])
    dst_p = jnp.concatenate([edge_index[1], jnp.full((pad,), N, jnp.int32)])
    w_p = jnp.concatenate([edge_weight, jnp.zeros((pad,), jnp.float32)])
    src2d = src_p.reshape(ROWS2D, GROUP)
    dst2d = dst_p.reshape(ROWS2D, GROUP)
    xtab = jnp.concatenate(
        [x, jnp.ones((N, 1), jnp.float32), jnp.zeros((N, 3), jnp.float32)],
        axis=1)
    zeros = jnp.zeros((R, 8), jnp.float32)

    part = _sc_aggregate(xtab, src2d, dst2d, w_p, zeros)

    out, h_t, c_t = _tc_dense(
        x, part, h_0, c_0, ggc_w,
        gru_wi.T, gru_wh.T, gru_bi.reshape(1, 12), gru_bh.reshape(1, 12),
        lstm_wi.T, lstm_wh.T, (lstm_bi + lstm_bh).reshape(1, 128),
        lin_w, lin_b.reshape(1, 1))
    return (out, h_t, c_t)


# speedup vs baseline: 27.7500x; 1.0115x over previous
import functools

import jax
import jax.numpy as jnp
from jax import lax
from jax.experimental import pallas as pl
from jax.experimental.pallas import tpu as pltpu
from jax.experimental.pallas import tpu_sc as plsc

N = 100000
E = 6400000
EPAD = 6553600
GROUP = 128
GROUPS_PER_CHUNK = 8
CHUNK = GROUP * GROUPS_PER_CHUNK
CHUNKS_PER_TILE = EPAD // (32 * CHUNK)
ROWS2D = EPAD // GROUP
R = 100352
RSLICE = R // 16


NBUF = 4


def _sc_body(xtab, src2d, dst2d, wflat, zeros, out, table, vsrc, vdst, vw, rows,
             gsem, ssem, isem):
    cid = lax.axis_index("c")
    sid = lax.axis_index("s")
    wid = cid * 16 + sid

    @pl.when(sid == 0)
    def _zero():
        pltpu.sync_copy(zeros, table)

    plsc.subcore_barrier()

    iota = lax.broadcasted_iota(jnp.int32, (16,), 0)
    ci4 = iota & 3
    r4 = iota >> 2

    def fire_idx(c, b):
        row0 = wid * (CHUNKS_PER_TILE * GROUPS_PER_CHUNK) + c * GROUPS_PER_CHUNK
        e0 = wid * (CHUNKS_PER_TILE * CHUNK) + c * CHUNK
        pltpu.async_copy(src2d.at[pl.ds(row0, GROUPS_PER_CHUNK)], vsrc.at[b], isem)
        pltpu.async_copy(dst2d.at[pl.ds(row0, GROUPS_PER_CHUNK)], vdst.at[b], isem)
        pltpu.async_copy(wflat.at[pl.ds(e0, CHUNK)], vw.at[b], isem)

    def wait_idx(b):
        for _ in range(3):
            pltpu.make_async_copy(src2d.at[pl.ds(0, GROUPS_PER_CHUNK)],
                                  vsrc.at[b], isem).wait()

    def fire_gathers(b):
        for g in range(GROUPS_PER_CHUNK):
            pltpu.async_copy(xtab.at[vsrc.at[b, g]], rows.at[b, g], gsem)

    def wait_gathers(b):
        for g in range(GROUPS_PER_CHUNK):
            pltpu.make_async_copy(xtab.at[vsrc.at[b, g]], rows.at[b, g],
                                  gsem).wait()

    def fire_scatters(b):
        for g in range(GROUPS_PER_CHUNK):
            pltpu.async_copy(rows.at[b, g], table.at[vdst.at[b, g]], ssem,
                             add=True)

    def wait_scatters(b):
        for g in range(GROUPS_PER_CHUNK):
            pltpu.make_async_copy(rows.at[b, g], table.at[vdst.at[b, g]],
                                  ssem).wait()

    def mul(b):
        b_s = jnp.full((16,), b, jnp.int32)
        for g in range(GROUPS_PER_CHUNK):
            g_s = jnp.full((16,), g, jnp.int32)

            def _mb(i, carry):
                s = 32 * i
                for u in range(8):
                    ri = (s + 4 * u) + r4
                    wb = plsc.load_gather(vw, [b_s, g * GROUP + ri])
                    v = plsc.load_gather(rows, [b_s, g_s, ri, ci4])
                    plsc.store_scatter(rows, [b_s, g_s, ri, ci4], v * wb)
                return carry

            lax.fori_loop(0, GROUP // 32, _mb, 0)

    fire_idx(0, 0)
    wait_idx(0)
    fire_gathers(0)
    fire_idx(1, 1)

    def super_body(s, carry):
        for b in range(NBUF):
            c = NBUF * s + b
            b1 = (b + 1) % NBUF
            b2 = (b + 2) % NBUF

            @pl.when(c >= 2)
            def _():
                wait_scatters(b2)

            @pl.when(c + 2 < CHUNKS_PER_TILE)
            def _():
                fire_idx(c + 2, b2)

            wait_gathers(b)
            mul(b)

            @pl.when(c + 1 < CHUNKS_PER_TILE)
            def _():
                wait_idx(b1)
                fire_gathers(b1)

            fire_scatters(b)
        return carry

    lax.fori_loop(0, CHUNKS_PER_TILE // NBUF, super_body, 0)
    wait_scatters((CHUNKS_PER_TILE - 2) % NBUF)
    wait_scatters((CHUNKS_PER_TILE - 1) % NBUF)

    plsc.subcore_barrier()
    pltpu.sync_copy(table.at[pl.ds(sid * RSLICE, RSLICE)],
                    out.at[cid, pl.ds(sid * RSLICE, RSLICE)])


def _sc_aggregate(xtab, src2d, dst2d, wflat, zeros):
    mesh = plsc.VectorSubcoreMesh(core_axis_name="c", subcore_axis_name="s")
    return pl.kernel(
        _sc_body,
        out_type=jax.ShapeDtypeStruct((2, R, 8), jnp.float32),
        mesh=mesh,
        compiler_params=pltpu.CompilerParams(needs_layout_passes=False,
                                             use_tc_tiling_on_sc=False),
        scratch_types=[
            pltpu.VMEM_SHARED((R, 8), jnp.float32),
            pltpu.VMEM((NBUF, GROUPS_PER_CHUNK, GROUP), jnp.int32),
            pltpu.VMEM((NBUF, GROUPS_PER_CHUNK, GROUP), jnp.int32),
            pltpu.VMEM((NBUF, CHUNK), jnp.float32),
            pltpu.VMEM((NBUF, GROUPS_PER_CHUNK, GROUP, 8), jnp.float32),
            pltpu.SemaphoreType.DMA,
            pltpu.SemaphoreType.DMA,
            pltpu.SemaphoreType.DMA,
        ],
    )(xtab, src2d, dst2d, wflat, zeros)


BLK = 4000


def _tc_body(x_r, p_r, h0_r, c0_r, ggc_r, wi_r, wh_r, bi_r, bh_r,
             lwi_r, lwh_r, lb_r, lin_r, linb_r, out_r, ht_r, ct_r):
    p = p_r[...]
    aggx = p[0, :, 0:4] + p[1, :, 0:4]
    cnt = p[0, :, 4:5] + p[1, :, 4:5]
    agg = jnp.dot(aggx, ggc_r[...], preferred_element_type=jnp.float32)
    agg = agg / jnp.maximum(cnt, 1.0)
    xb = x_r[...]
    gi = jnp.dot(agg, wi_r[...], preferred_element_type=jnp.float32) + bi_r[...]
    gh = jnp.dot(xb, wh_r[...], preferred_element_type=jnp.float32) + bh_r[...]
    r = jax.nn.sigmoid(gi[:, 0:4] + gh[:, 0:4])
    z = jax.nn.sigmoid(gi[:, 4:8] + gh[:, 4:8])
    nc = jnp.tanh(gi[:, 8:12] + r * gh[:, 8:12])
    hc = (1.0 - z) * nc + z * xb
    gates = (jnp.dot(hc, lwi_r[...], preferred_element_type=jnp.float32)
             + jnp.dot(h0_r[...], lwh_r[...], preferred_element_type=jnp.float32)
             + lb_r[...])
    i_t = jax.nn.sigmoid(gates[:, 0:32])
    f_t = jax.nn.sigmoid(gates[:, 32:64])
    g_t = jnp.tanh(gates[:, 64:96])
    o_t = jax.nn.sigmoid(gates[:, 96:128])
    c_t = f_t * c0_r[...] + i_t * g_t
    h_t = o_t * jnp.tanh(c_t)
    ct_r[...] = c_t
    ht_r[...] = h_t
    out_r[...] = (jnp.sum(jax.nn.relu(h_t) * lin_r[...], axis=1, keepdims=True)
                  + linb_r[...])


def _tc_dense(x, part, h_0, c_0, ggc_w, gru_wiT, gru_whT, gru_bi2, gru_bh2,
              lstm_wiT, lstm_whT, lstm_b2, lin_w, lin_b2):
    nblk = N // BLK
    full = lambda a: pl.BlockSpec(a.shape, lambda i: (0,) * a.ndim)
    return pl.pallas_call(
        _tc_body,
        grid=(nblk,),
        in_specs=[
            pl.BlockSpec((BLK, 4), lambda i: (i, 0)),
            pl.BlockSpec((2, BLK, 8), lambda i: (0, i, 0)),
            pl.BlockSpec((BLK, 32), lambda i: (i, 0)),
            pl.BlockSpec((BLK, 32), lambda i: (i, 0)),
            full(ggc_w), full(gru_wiT), full(gru_whT), full(gru_bi2),
            full(gru_bh2), full(lstm_wiT), full(lstm_whT), full(lstm_b2),
            full(lin_w), full(lin_b2),
        ],
        out_specs=[
            pl.BlockSpec((BLK, 1), lambda i: (i, 0)),
            pl.BlockSpec((BLK, 32), lambda i: (i, 0)),
            pl.BlockSpec((BLK, 32), lambda i: (i, 0)),
        ],
        out_shape=[
            jax.ShapeDtypeStruct((N, 1), jnp.float32),
            jax.ShapeDtypeStruct((N, 32), jnp.float32),
            jax.ShapeDtypeStruct((N, 32), jnp.float32),
        ],
    )(x, part, h_0, c_0, ggc_w, gru_wiT, gru_whT, gru_bi2, gru_bh2,
      lstm_wiT, lstm_whT, lstm_b2, lin_w, lin_b2)


def kernel(x, edge_index, edge_weight, h_0, c_0, ggc_w, gru_wi, gru_wh,
           gru_bi, gru_bh, lstm_wi, lstm_wh, lstm_bi, lstm_bh, lin_w, lin_b):
    pad = EPAD - E
    src_p = jnp.concatenate([edge_index[0], jnp.zeros((pad,), jnp.int32)])
    pad_dst = N + (jnp.arange(pad, dtype=jnp.int32) & 127)
    dst_p = jnp.concatenate([edge_index[1], pad_dst])
    w_p = jnp.concatenate([edge_weight, jnp.zeros((pad,), jnp.float32)])
    src2d = src_p.reshape(ROWS2D, GROUP)
    dst2d = dst_p.reshape(ROWS2D, GROUP)
    xtab = jnp.concatenate(
        [x, jnp.ones((N, 1), jnp.float32), jnp.zeros((N, 3), jnp.float32)],
        axis=1)
    zeros = jnp.zeros((R, 8), jnp.float32)

    part = _sc_aggregate(xtab, src2d, dst2d, w_p, zeros)

    out, h_t, c_t = _tc_dense(
        x, part, h_0, c_0, ggc_w,
        gru_wi.T, gru_wh.T, gru_bi.reshape(1, 12), gru_bh.reshape(1, 12),
        lstm_wi.T, lstm_wh.T, (lstm_bi + lstm_bh).reshape(1, 128),
        lin_w, lin_b.reshape(1, 1))
    return (out, h_t, c_t)


# speedup vs baseline: 29.3934x; 1.0592x over previous
import functools

import jax
import jax.numpy as jnp
from jax import lax
from jax.experimental import pallas as pl
from jax.experimental.pallas import tpu as pltpu
from jax.experimental.pallas import tpu_sc as plsc

N = 100000
E = 6400000
EPAD = 6553600
GROUP = 512
GROUPS_PER_CHUNK = 2
CHUNK = GROUP * GROUPS_PER_CHUNK
CHUNKS_PER_TILE = EPAD // (32 * CHUNK)
ROWS2D = EPAD // GROUP
R = 100352
RSLICE = R // 16


NBUF = 4


def _sc_body(xtab, src2d, dst2d, wflat, zeros, out, table, vsrc, vdst, vw, rows,
             gsem, ssem, isem):
    cid = lax.axis_index("c")
    sid = lax.axis_index("s")
    wid = cid * 16 + sid

    @pl.when(sid == 0)
    def _zero():
        pltpu.sync_copy(zeros, table)

    plsc.subcore_barrier()

    iota = lax.broadcasted_iota(jnp.int32, (16,), 0)
    ci4 = iota & 3
    r4 = iota >> 2

    def fire_idx(c, b):
        row0 = wid * (CHUNKS_PER_TILE * GROUPS_PER_CHUNK) + c * GROUPS_PER_CHUNK
        e0 = wid * (CHUNKS_PER_TILE * CHUNK) + c * CHUNK
        pltpu.async_copy(src2d.at[pl.ds(row0, GROUPS_PER_CHUNK)], vsrc.at[b], isem)
        pltpu.async_copy(dst2d.at[pl.ds(row0, GROUPS_PER_CHUNK)], vdst.at[b], isem)
        pltpu.async_copy(wflat.at[pl.ds(e0, CHUNK)], vw.at[b], isem)

    def wait_idx(b):
        for _ in range(3):
            pltpu.make_async_copy(src2d.at[pl.ds(0, GROUPS_PER_CHUNK)],
                                  vsrc.at[b], isem).wait()

    def fire_gathers(b):
        for g in range(GROUPS_PER_CHUNK):
            pltpu.async_copy(xtab.at[vsrc.at[b, g]], rows.at[b, g], gsem)

    def wait_gathers(b):
        for g in range(GROUPS_PER_CHUNK):
            pltpu.make_async_copy(xtab.at[vsrc.at[b, g]], rows.at[b, g],
                                  gsem).wait()

    def fire_scatters(b):
        for g in range(GROUPS_PER_CHUNK):
            pltpu.async_copy(rows.at[b, g], table.at[vdst.at[b, g]], ssem,
                             add=True)

    def wait_scatters(b):
        for g in range(GROUPS_PER_CHUNK):
            pltpu.make_async_copy(rows.at[b, g], table.at[vdst.at[b, g]],
                                  ssem).wait()

    def mul(b):
        b_s = jnp.full((16,), b, jnp.int32)
        for g in range(GROUPS_PER_CHUNK):
            g_s = jnp.full((16,), g, jnp.int32)

            def _mb(i, carry):
                s = 32 * i
                for u in range(8):
                    ri = (s + 4 * u) + r4
                    wb = plsc.load_gather(vw, [b_s, g * GROUP + ri])
                    v = plsc.load_gather(rows, [b_s, g_s, ri, ci4])
                    plsc.store_scatter(rows, [b_s, g_s, ri, ci4], v * wb)
                return carry

            lax.fori_loop(0, GROUP // 32, _mb, 0)

    fire_idx(0, 0)
    wait_idx(0)
    fire_gathers(0)
    fire_idx(1, 1)

    def super_body(s, carry):
        for b in range(NBUF):
            c = NBUF * s + b
            b1 = (b + 1) % NBUF
            b2 = (b + 2) % NBUF

            @pl.when(c >= 2)
            def _():
                wait_scatters(b2)

            @pl.when(c + 2 < CHUNKS_PER_TILE)
            def _():
                fire_idx(c + 2, b2)

            wait_gathers(b)
            mul(b)

            @pl.when(c + 1 < CHUNKS_PER_TILE)
            def _():
                wait_idx(b1)
                fire_gathers(b1)

            fire_scatters(b)
        return carry

    lax.fori_loop(0, CHUNKS_PER_TILE // NBUF, super_body, 0)
    wait_scatters((CHUNKS_PER_TILE - 2) % NBUF)
    wait_scatters((CHUNKS_PER_TILE - 1) % NBUF)

    plsc.subcore_barrier()
    pltpu.sync_copy(table.at[pl.ds(sid * RSLICE, RSLICE)],
                    out.at[cid, pl.ds(sid * RSLICE, RSLICE)])


def _sc_aggregate(xtab, src2d, dst2d, wflat, zeros):
    mesh = plsc.VectorSubcoreMesh(core_axis_name="c", subcore_axis_name="s")
    return pl.kernel(
        _sc_body,
        out_type=jax.ShapeDtypeStruct((2, R, 8), jnp.float32),
        mesh=mesh,
        compiler_params=pltpu.CompilerParams(needs_layout_passes=False,
                                             use_tc_tiling_on_sc=False),
        scratch_types=[
            pltpu.VMEM_SHARED((R, 8), jnp.float32),
            pltpu.VMEM((NBUF, GROUPS_PER_CHUNK, GROUP), jnp.int32),
            pltpu.VMEM((NBUF, GROUPS_PER_CHUNK, GROUP), jnp.int32),
            pltpu.VMEM((NBUF, CHUNK), jnp.float32),
            pltpu.VMEM((NBUF, GROUPS_PER_CHUNK, GROUP, 8), jnp.float32),
            pltpu.SemaphoreType.DMA,
            pltpu.SemaphoreType.DMA,
            pltpu.SemaphoreType.DMA,
        ],
    )(xtab, src2d, dst2d, wflat, zeros)


BLK = 4000


def _tc_body(x_r, p_r, h0_r, c0_r, ggc_r, wi_r, wh_r, bi_r, bh_r,
             lwi_r, lwh_r, lb_r, lin_r, linb_r, out_r, ht_r, ct_r):
    p = p_r[...]
    aggx = p[0, :, 0:4] + p[1, :, 0:4]
    cnt = p[0, :, 4:5] + p[1, :, 4:5]
    agg = jnp.dot(aggx, ggc_r[...], preferred_element_type=jnp.float32)
    agg = agg / jnp.maximum(cnt, 1.0)
    xb = x_r[...]
    gi = jnp.dot(agg, wi_r[...], preferred_element_type=jnp.float32) + bi_r[...]
    gh = jnp.dot(xb, wh_r[...], preferred_element_type=jnp.float32) + bh_r[...]
    r = jax.nn.sigmoid(gi[:, 0:4] + gh[:, 0:4])
    z = jax.nn.sigmoid(gi[:, 4:8] + gh[:, 4:8])
    nc = jnp.tanh(gi[:, 8:12] + r * gh[:, 8:12])
    hc = (1.0 - z) * nc + z * xb
    gates = (jnp.dot(hc, lwi_r[...], preferred_element_type=jnp.float32)
             + jnp.dot(h0_r[...], lwh_r[...], preferred_element_type=jnp.float32)
             + lb_r[...])
    i_t = jax.nn.sigmoid(gates[:, 0:32])
    f_t = jax.nn.sigmoid(gates[:, 32:64])
    g_t = jnp.tanh(gates[:, 64:96])
    o_t = jax.nn.sigmoid(gates[:, 96:128])
    c_t = f_t * c0_r[...] + i_t * g_t
    h_t = o_t * jnp.tanh(c_t)
    ct_r[...] = c_t
    ht_r[...] = h_t
    out_r[...] = (jnp.sum(jax.nn.relu(h_t) * lin_r[...], axis=1, keepdims=True)
                  + linb_r[...])


def _tc_dense(x, part, h_0, c_0, ggc_w, gru_wiT, gru_whT, gru_bi2, gru_bh2,
              lstm_wiT, lstm_whT, lstm_b2, lin_w, lin_b2):
    nblk = N // BLK
    full = lambda a: pl.BlockSpec(a.shape, lambda i: (0,) * a.ndim)
    return pl.pallas_call(
        _tc_body,
        grid=(nblk,),
        in_specs=[
            pl.BlockSpec((BLK, 4), lambda i: (i, 0)),
            pl.BlockSpec((2, BLK, 8), lambda i: (0, i, 0)),
            pl.BlockSpec((BLK, 32), lambda i: (i, 0)),
            pl.BlockSpec((BLK, 32), lambda i: (i, 0)),
            full(ggc_w), full(gru_wiT), full(gru_whT), full(gru_bi2),
            full(gru_bh2), full(lstm_wiT), full(lstm_whT), full(lstm_b2),
            full(lin_w), full(lin_b2),
        ],
        out_specs=[
            pl.BlockSpec((BLK, 1), lambda i: (i, 0)),
            pl.BlockSpec((BLK, 32), lambda i: (i, 0)),
            pl.BlockSpec((BLK, 32), lambda i: (i, 0)),
        ],
        out_shape=[
            jax.ShapeDtypeStruct((N, 1), jnp.float32),
            jax.ShapeDtypeStruct((N, 32), jnp.float32),
            jax.ShapeDtypeStruct((N, 32), jnp.float32),
        ],
    )(x, part, h_0, c_0, ggc_w, gru_wiT, gru_whT, gru_bi2, gru_bh2,
      lstm_wiT, lstm_whT, lstm_b2, lin_w, lin_b2)


def kernel(x, edge_index, edge_weight, h_0, c_0, ggc_w, gru_wi, gru_wh,
           gru_bi, gru_bh, lstm_wi, lstm_wh, lstm_bi, lstm_bh, lin_w, lin_b):
    pad = EPAD - E
    src_p = jnp.concatenate([edge_index[0], jnp.zeros((pad,), jnp.int32)])
    pad_dst = N + (jnp.arange(pad, dtype=jnp.int32) & 127)
    dst_p = jnp.concatenate([edge_index[1], pad_dst])
    w_p = jnp.concatenate([edge_weight, jnp.zeros((pad,), jnp.float32)])
    src2d = src_p.reshape(ROWS2D, GROUP)
    dst2d = dst_p.reshape(ROWS2D, GROUP)
    xtab = jnp.concatenate(
        [x, jnp.ones((N, 1), jnp.float32), jnp.zeros((N, 3), jnp.float32)],
        axis=1)
    zeros = jnp.zeros((R, 8), jnp.float32)

    part = _sc_aggregate(xtab, src2d, dst2d, w_p, zeros)

    out, h_t, c_t = _tc_dense(
        x, part, h_0, c_0, ggc_w,
        gru_wi.T, gru_wh.T, gru_bi.reshape(1, 12), gru_bh.reshape(1, 12),
        lstm_wi.T, lstm_wh.T, (lstm_bi + lstm_bh).reshape(1, 128),
        lin_w, lin_b.reshape(1, 1))
    return (out, h_t, c_t)


# speedup vs baseline: 29.4287x; 1.0012x over previous
import functools

import jax
import jax.numpy as jnp
from jax import lax
from jax.experimental import pallas as pl
from jax.experimental.pallas import tpu as pltpu
from jax.experimental.pallas import tpu_sc as plsc

N = 100000
E = 6400000
EPAD = 6553600
GROUP = 1024
GROUPS_PER_CHUNK = 1
CHUNK = GROUP * GROUPS_PER_CHUNK
CHUNKS_PER_TILE = EPAD // (32 * CHUNK)
ROWS2D = EPAD // GROUP
R = 100352
RSLICE = R // 16


NBUF = 4


def _sc_body(xtab, src2d, dst2d, wflat, zeros, out, table, vsrc, vdst, vw, rows,
             gsem, ssem, isem):
    cid = lax.axis_index("c")
    sid = lax.axis_index("s")
    wid = cid * 16 + sid

    @pl.when(sid == 0)
    def _zero():
        pltpu.sync_copy(zeros, table)

    plsc.subcore_barrier()

    iota = lax.broadcasted_iota(jnp.int32, (16,), 0)
    ci4 = iota & 3
    r4 = iota >> 2

    def fire_idx(c, b):
        row0 = wid * (CHUNKS_PER_TILE * GROUPS_PER_CHUNK) + c * GROUPS_PER_CHUNK
        e0 = wid * (CHUNKS_PER_TILE * CHUNK) + c * CHUNK
        pltpu.async_copy(src2d.at[pl.ds(row0, GROUPS_PER_CHUNK)], vsrc.at[b], isem)
        pltpu.async_copy(dst2d.at[pl.ds(row0, GROUPS_PER_CHUNK)], vdst.at[b], isem)
        pltpu.async_copy(wflat.at[pl.ds(e0, CHUNK)], vw.at[b], isem)

    def wait_idx(b):
        for _ in range(3):
            pltpu.make_async_copy(src2d.at[pl.ds(0, GROUPS_PER_CHUNK)],
                                  vsrc.at[b], isem).wait()

    def fire_gathers(b):
        for g in range(GROUPS_PER_CHUNK):
            pltpu.async_copy(xtab.at[vsrc.at[b, g]], rows.at[b, g], gsem)

    def wait_gathers(b):
        for g in range(GROUPS_PER_CHUNK):
            pltpu.make_async_copy(xtab.at[vsrc.at[b, g]], rows.at[b, g],
                                  gsem).wait()

    def fire_scatters(b):
        for g in range(GROUPS_PER_CHUNK):
            pltpu.async_copy(rows.at[b, g], table.at[vdst.at[b, g]], ssem,
                             add=True)

    def wait_scatters(b):
        for g in range(GROUPS_PER_CHUNK):
            pltpu.make_async_copy(rows.at[b, g], table.at[vdst.at[b, g]],
                                  ssem).wait()

    def mul(b):
        b_s = jnp.full((16,), b, jnp.int32)
        for g in range(GROUPS_PER_CHUNK):
            g_s = jnp.full((16,), g, jnp.int32)

            def _mb(i, carry):
                s = 32 * i
                for u in range(8):
                    ri = (s + 4 * u) + r4
                    wb = plsc.load_gather(vw, [b_s, g * GROUP + ri])
                    v = plsc.load_gather(rows, [b_s, g_s, ri, ci4])
                    plsc.store_scatter(rows, [b_s, g_s, ri, ci4], v * wb)
                return carry

            lax.fori_loop(0, GROUP // 32, _mb, 0)

    fire_idx(0, 0)
    wait_idx(0)
    fire_gathers(0)
    fire_idx(1, 1)

    def super_body(s, carry):
        for b in range(NBUF):
            c = NBUF * s + b
            b1 = (b + 1) % NBUF
            b2 = (b + 2) % NBUF

            @pl.when(c >= 2)
            def _():
                wait_scatters(b2)

            @pl.when(c + 2 < CHUNKS_PER_TILE)
            def _():
                fire_idx(c + 2, b2)

            wait_gathers(b)
            mul(b)

            @pl.when(c + 1 < CHUNKS_PER_TILE)
            def _():
                wait_idx(b1)
                fire_gathers(b1)

            fire_scatters(b)
        return carry

    lax.fori_loop(0, CHUNKS_PER_TILE // NBUF, super_body, 0)
    wait_scatters((CHUNKS_PER_TILE - 2) % NBUF)
    wait_scatters((CHUNKS_PER_TILE - 1) % NBUF)

    plsc.subcore_barrier()
    pltpu.sync_copy(table.at[pl.ds(sid * RSLICE, RSLICE)],
                    out.at[cid, pl.ds(sid * RSLICE, RSLICE)])


def _sc_aggregate(xtab, src2d, dst2d, wflat, zeros):
    mesh = plsc.VectorSubcoreMesh(core_axis_name="c", subcore_axis_name="s")
    return pl.kernel(
        _sc_body,
        out_type=jax.ShapeDtypeStruct((2, R, 8), jnp.float32),
        mesh=mesh,
        compiler_params=pltpu.CompilerParams(needs_layout_passes=False,
                                             use_tc_tiling_on_sc=False),
        scratch_types=[
            pltpu.VMEM_SHARED((R, 8), jnp.float32),
            pltpu.VMEM((NBUF, GROUPS_PER_CHUNK, GROUP), jnp.int32),
            pltpu.VMEM((NBUF, GROUPS_PER_CHUNK, GROUP), jnp.int32),
            pltpu.VMEM((NBUF, CHUNK), jnp.float32),
            pltpu.VMEM((NBUF, GROUPS_PER_CHUNK, GROUP, 8), jnp.float32),
            pltpu.SemaphoreType.DMA,
            pltpu.SemaphoreType.DMA,
            pltpu.SemaphoreType.DMA,
        ],
    )(xtab, src2d, dst2d, wflat, zeros)


BLK = 4000


def _tc_body(x_r, p_r, h0_r, c0_r, ggc_r, wi_r, wh_r, bi_r, bh_r,
             lwi_r, lwh_r, lb_r, lin_r, linb_r, out_r, ht_r, ct_r):
    p = p_r[...]
    aggx = p[0, :, 0:4] + p[1, :, 0:4]
    cnt = p[0, :, 4:5] + p[1, :, 4:5]
    agg = jnp.dot(aggx, ggc_r[...], preferred_element_type=jnp.float32)
    agg = agg / jnp.maximum(cnt, 1.0)
    xb = x_r[...]
    gi = jnp.dot(agg, wi_r[...], preferred_element_type=jnp.float32) + bi_r[...]
    gh = jnp.dot(xb, wh_r[...], preferred_element_type=jnp.float32) + bh_r[...]
    r = jax.nn.sigmoid(gi[:, 0:4] + gh[:, 0:4])
    z = jax.nn.sigmoid(gi[:, 4:8] + gh[:, 4:8])
    nc = jnp.tanh(gi[:, 8:12] + r * gh[:, 8:12])
    hc = (1.0 - z) * nc + z * xb
    gates = (jnp.dot(hc, lwi_r[...], preferred_element_type=jnp.float32)
             + jnp.dot(h0_r[...], lwh_r[...], preferred_element_type=jnp.float32)
             + lb_r[...])
    i_t = jax.nn.sigmoid(gates[:, 0:32])
    f_t = jax.nn.sigmoid(gates[:, 32:64])
    g_t = jnp.tanh(gates[:, 64:96])
    o_t = jax.nn.sigmoid(gates[:, 96:128])
    c_t = f_t * c0_r[...] + i_t * g_t
    h_t = o_t * jnp.tanh(c_t)
    ct_r[...] = c_t
    ht_r[...] = h_t
    out_r[...] = (jnp.sum(jax.nn.relu(h_t) * lin_r[...], axis=1, keepdims=True)
                  + linb_r[...])


def _tc_dense(x, part, h_0, c_0, ggc_w, gru_wiT, gru_whT, gru_bi2, gru_bh2,
              lstm_wiT, lstm_whT, lstm_b2, lin_w, lin_b2):
    nblk = N // BLK
    full = lambda a: pl.BlockSpec(a.shape, lambda i: (0,) * a.ndim)
    return pl.pallas_call(
        _tc_body,
        grid=(nblk,),
        in_specs=[
            pl.BlockSpec((BLK, 4), lambda i: (i, 0)),
            pl.BlockSpec((2, BLK, 8), lambda i: (0, i, 0)),
            pl.BlockSpec((BLK, 32), lambda i: (i, 0)),
            pl.BlockSpec((BLK, 32), lambda i: (i, 0)),
            full(ggc_w), full(gru_wiT), full(gru_whT), full(gru_bi2),
            full(gru_bh2), full(lstm_wiT), full(lstm_whT), full(lstm_b2),
            full(lin_w), full(lin_b2),
        ],
        out_specs=[
            pl.BlockSpec((BLK, 1), lambda i: (i, 0)),
            pl.BlockSpec((BLK, 32), lambda i: (i, 0)),
            pl.BlockSpec((BLK, 32), lambda i: (i, 0)),
        ],
        out_shape=[
            jax.ShapeDtypeStruct((N, 1), jnp.float32),
            jax.ShapeDtypeStruct((N, 32), jnp.float32),
            jax.ShapeDtypeStruct((N, 32), jnp.float32),
        ],
    )(x, part, h_0, c_0, ggc_w, gru_wiT, gru_whT, gru_bi2, gru_bh2,
      lstm_wiT, lstm_whT, lstm_b2, lin_w, lin_b2)


def kernel(x, edge_index, edge_weight, h_0, c_0, ggc_w, gru_wi, gru_wh,
           gru_bi, gru_bh, lstm_wi, lstm_wh, lstm_bi, lstm_bh, lin_w, lin_b):
    pad = EPAD - E
    src_p = jnp.concatenate([edge_index[0], jnp.zeros((pad,), jnp.int32)])
    pad_dst = N + (jnp.arange(pad, dtype=jnp.int32) & 127)
    dst_p = jnp.concatenate([edge_index[1], pad_dst])
    w_p = jnp.concatenate([edge_weight, jnp.zeros((pad,), jnp.float32)])
    src2d = src_p.reshape(ROWS2D, GROUP)
    dst2d = dst_p.reshape(ROWS2D, GROUP)
    xtab = jnp.concatenate(
        [x, jnp.ones((N, 1), jnp.float32), jnp.zeros((N, 3), jnp.float32)],
        axis=1)
    zeros = jnp.zeros((R, 8), jnp.float32)

    part = _sc_aggregate(xtab, src2d, dst2d, w_p, zeros)

    out, h_t, c_t = _tc_dense(
        x, part, h_0, c_0, ggc_w,
        gru_wi.T, gru_wh.T, gru_bi.reshape(1, 12), gru_bh.reshape(1, 12),
        lstm_wi.T, lstm_wh.T, (lstm_bi + lstm_bh).reshape(1, 128),
        lin_w, lin_b.reshape(1, 1))
    return (out, h_t, c_t)


# speedup vs baseline: 30.1342x; 1.0240x over previous
import functools

import jax
import jax.numpy as jnp
from jax import lax
from jax.experimental import pallas as pl
from jax.experimental.pallas import tpu as pltpu
from jax.experimental.pallas import tpu_sc as plsc

N = 100000
E = 6400000
EPAD = 6553600
GROUP = 1024
GROUPS_PER_CHUNK = 1
CHUNK = GROUP * GROUPS_PER_CHUNK
CHUNKS_PER_TILE = EPAD // (32 * CHUNK)
ROWS2D = EPAD // GROUP
R = 100352
RSLICE = R // 16


NBUF = 4


def _sc_body(xtab, src2d, dst2d, wflat, zeros, out, table, vsrc, vdst, vw, rows,
             gsem, ssem, isem):
    cid = lax.axis_index("c")
    sid = lax.axis_index("s")
    wid = cid * 16 + sid

    @pl.when(sid == 0)
    def _zero():
        pltpu.sync_copy(zeros, table)

    plsc.subcore_barrier()

    iota = lax.broadcasted_iota(jnp.int32, (16,), 0)
    ci4 = iota & 3
    r4 = iota >> 2

    def fire_idx(c, b):
        row0 = wid * (CHUNKS_PER_TILE * GROUPS_PER_CHUNK) + c * GROUPS_PER_CHUNK
        e0 = wid * (CHUNKS_PER_TILE * CHUNK) + c * CHUNK
        pltpu.async_copy(src2d.at[pl.ds(row0, GROUPS_PER_CHUNK)], vsrc.at[b], isem)
        pltpu.async_copy(dst2d.at[pl.ds(row0, GROUPS_PER_CHUNK)], vdst.at[b], isem)
        pltpu.async_copy(wflat.at[pl.ds(e0, CHUNK)], vw.at[b], isem)

    def wait_idx(b):
        for _ in range(3):
            pltpu.make_async_copy(src2d.at[pl.ds(0, GROUPS_PER_CHUNK)],
                                  vsrc.at[b], isem).wait()

    def fire_gathers(b):
        for g in range(GROUPS_PER_CHUNK):
            pltpu.async_copy(xtab.at[vsrc.at[b, g]], rows.at[b, g], gsem)

    def wait_gathers(b):
        for g in range(GROUPS_PER_CHUNK):
            pltpu.make_async_copy(xtab.at[vsrc.at[b, g]], rows.at[b, g],
                                  gsem).wait()

    def fire_scatters(b):
        for g in range(GROUPS_PER_CHUNK):
            pltpu.async_copy(rows.at[b, g], table.at[vdst.at[b, g]], ssem,
                             add=True)

    def wait_scatters(b):
        for g in range(GROUPS_PER_CHUNK):
            pltpu.make_async_copy(rows.at[b, g], table.at[vdst.at[b, g]],
                                  ssem).wait()

    def mul(b):
        b_s = jnp.full((16,), b, jnp.int32)
        for g in range(GROUPS_PER_CHUNK):
            g_s = jnp.full((16,), g, jnp.int32)

            def _mb(i, carry):
                s = 32 * i
                for u in range(8):
                    ri = (s + 4 * u) + r4
                    wb = plsc.load_gather(vw, [b_s, g * GROUP + ri])
                    v = plsc.load_gather(rows, [b_s, g_s, ri, ci4])
                    plsc.store_scatter(rows, [b_s, g_s, ri, ci4], v * wb)
                return carry

            lax.fori_loop(0, GROUP // 32, _mb, 0)

    fire_idx(0, 0)
    wait_idx(0)
    fire_gathers(0)
    fire_idx(1, 1)

    def super_body(s, carry):
        for b in range(NBUF):
            c = NBUF * s + b
            b1 = (b + 1) % NBUF
            b2 = (b + 2) % NBUF

            @pl.when(c >= 2)
            def _():
                wait_scatters(b2)

            @pl.when(c + 2 < CHUNKS_PER_TILE)
            def _():
                fire_idx(c + 2, b2)

            wait_gathers(b)
            mul(b)

            @pl.when(c + 1 < CHUNKS_PER_TILE)
            def _():
                wait_idx(b1)
                fire_gathers(b1)

            fire_scatters(b)
        return carry

    lax.fori_loop(0, CHUNKS_PER_TILE // NBUF, super_body, 0)
    wait_scatters((CHUNKS_PER_TILE - 2) % NBUF)
    wait_scatters((CHUNKS_PER_TILE - 1) % NBUF)

    plsc.subcore_barrier()
    pltpu.sync_copy(table.at[pl.ds(sid * RSLICE, RSLICE)],
                    out.at[cid, pl.ds(sid * RSLICE, RSLICE)])


def _sc_aggregate(xtab, src2d, dst2d, wflat, zeros):
    mesh = plsc.VectorSubcoreMesh(core_axis_name="c", subcore_axis_name="s")
    return pl.kernel(
        _sc_body,
        out_type=jax.ShapeDtypeStruct((2, R, 8), jnp.float32),
        mesh=mesh,
        compiler_params=pltpu.CompilerParams(needs_layout_passes=False,
                                             use_tc_tiling_on_sc=False),
        scratch_types=[
            pltpu.VMEM_SHARED((R, 8), jnp.float32),
            pltpu.VMEM((NBUF, GROUPS_PER_CHUNK, GROUP), jnp.int32),
            pltpu.VMEM((NBUF, GROUPS_PER_CHUNK, GROUP), jnp.int32),
            pltpu.VMEM((NBUF, CHUNK), jnp.float32),
            pltpu.VMEM((NBUF, GROUPS_PER_CHUNK, GROUP, 8), jnp.float32),
            pltpu.SemaphoreType.DMA,
            pltpu.SemaphoreType.DMA,
            pltpu.SemaphoreType.DMA,
        ],
    )(xtab, src2d, dst2d, wflat, zeros)


BLK = 4000


def _dot(a, b):
    return jnp.dot(a, b, preferred_element_type=jnp.float32)


def _tc_body(x_r, p_r, h0_r, c0_r, ggc_r,
             wir_r, whr_r, br_r, wiz_r, whz_r, bz_r,
             win_r, whn_r, bin_r, bhn_r,
             wii_r, whi_r, bi_r, wif_r, whf_r, bf_r,
             wig_r, whg_r, bg_r, wio_r, who_r, bo_r,
             lin_r, linb_r, out_r, ht_r, ct_r):
    p = p_r[...]
    aggx = p[0, :, 0:4] + p[1, :, 0:4]
    cnt = p[0, :, 4:5] + p[1, :, 4:5]
    agg = _dot(aggx, ggc_r[...]) / jnp.maximum(cnt, 1.0)
    xb = x_r[...]
    h0 = h0_r[...]
    r = jax.nn.sigmoid(_dot(agg, wir_r[...]) + _dot(xb, whr_r[...]) + br_r[...])
    z = jax.nn.sigmoid(_dot(agg, wiz_r[...]) + _dot(xb, whz_r[...]) + bz_r[...])
    nc = jnp.tanh(_dot(agg, win_r[...]) + bin_r[...]
                  + r * (_dot(xb, whn_r[...]) + bhn_r[...]))
    hc = (1.0 - z) * nc + z * xb
    i_t = jax.nn.sigmoid(_dot(hc, wii_r[...]) + _dot(h0, whi_r[...]) + bi_r[...])
    f_t = jax.nn.sigmoid(_dot(hc, wif_r[...]) + _dot(h0, whf_r[...]) + bf_r[...])
    g_t = jnp.tanh(_dot(hc, wig_r[...]) + _dot(h0, whg_r[...]) + bg_r[...])
    o_t = jax.nn.sigmoid(_dot(hc, wio_r[...]) + _dot(h0, who_r[...]) + bo_r[...])
    c_t = f_t * c0_r[...] + i_t * g_t
    h_t = o_t * jnp.tanh(c_t)
    ct_r[...] = c_t
    ht_r[...] = h_t
    out_r[...] = _dot(jax.nn.relu(h_t), lin_r[...]) + linb_r[...]


def _tc_dense(x, part, h_0, c_0, *weights):
    nblk = N // BLK
    full = lambda a: pl.BlockSpec(a.shape, lambda i: (0,) * a.ndim)
    return pl.pallas_call(
        _tc_body,
        grid=(nblk,),
        in_specs=[
            pl.BlockSpec((BLK, 4), lambda i: (i, 0)),
            pl.BlockSpec((2, BLK, 8), lambda i: (0, i, 0)),
            pl.BlockSpec((BLK, 32), lambda i: (i, 0)),
            pl.BlockSpec((BLK, 32), lambda i: (i, 0)),
        ] + [full(w) for w in weights],
        out_specs=[
            pl.BlockSpec((BLK, 1), lambda i: (i, 0)),
            pl.BlockSpec((BLK, 32), lambda i: (i, 0)),
            pl.BlockSpec((BLK, 32), lambda i: (i, 0)),
        ],
        out_shape=[
            jax.ShapeDtypeStruct((N, 1), jnp.float32),
            jax.ShapeDtypeStruct((N, 32), jnp.float32),
            jax.ShapeDtypeStruct((N, 32), jnp.float32),
        ],
    )(x, part, h_0, c_0, *weights)


def kernel(x, edge_index, edge_weight, h_0, c_0, ggc_w, gru_wi, gru_wh,
           gru_bi, gru_bh, lstm_wi, lstm_wh, lstm_bi, lstm_bh, lin_w, lin_b):
    pad = EPAD - E
    src_p = jnp.concatenate([edge_index[0], jnp.zeros((pad,), jnp.int32)])
    pad_dst = N + (jnp.arange(pad, dtype=jnp.int32) & 127)
    dst_p = jnp.concatenate([edge_index[1], pad_dst])
    w_p = jnp.concatenate([edge_weight, jnp.zeros((pad,), jnp.float32)])
    src2d = src_p.reshape(ROWS2D, GROUP)
    dst2d = dst_p.reshape(ROWS2D, GROUP)
    xtab = jnp.concatenate(
        [x, jnp.ones((N, 1), jnp.float32), jnp.zeros((N, 3), jnp.float32)],
        axis=1)
    zeros = jnp.zeros((R, 8), jnp.float32)

    part = _sc_aggregate(xtab, src2d, dst2d, w_p, zeros)

    gwi, gwh = gru_wi.T, gru_wh.T
    lwi, lwh = lstm_wi.T, lstm_wh.T
    gb = gru_bi + gru_bh
    lb = lstm_bi + lstm_bh
    weights = [
        ggc_w,
        gwi[:, 0:4], gwh[:, 0:4], gb[0:4].reshape(1, 4),
        gwi[:, 4:8], gwh[:, 4:8], gb[4:8].reshape(1, 4),
        gwi[:, 8:12], gwh[:, 8:12],
        gru_bi[8:12].reshape(1, 4), gru_bh[8:12].reshape(1, 4),
        lwi[:, 0:32], lwh[:, 0:32], lb[0:32].reshape(1, 32),
        lwi[:, 32:64], lwh[:, 32:64], lb[32:64].reshape(1, 32),
        lwi[:, 64:96], lwh[:, 64:96], lb[64:96].reshape(1, 32),
        lwi[:, 96:128], lwh[:, 96:128], lb[96:128].reshape(1, 32),
        lin_w.T, lin_b.reshape(1, 1),
    ]
    out, h_t, c_t = _tc_dense(x, part, h_0, c_0, *weights)
    return (out, h_t, c_t)


# speedup vs baseline: 31.3105x; 1.0390x over previous
import functools

import jax
import jax.numpy as jnp
from jax import lax
from jax.experimental import pallas as pl
from jax.experimental.pallas import tpu as pltpu
from jax.experimental.pallas import tpu_sc as plsc

N = 100000
E = 6400000
EPAD = 6553600
GROUP = 1024
GROUPS_PER_CHUNK = 1
CHUNK = GROUP * GROUPS_PER_CHUNK
NCK0 = 230
NCK1 = 170
ROWS2D = EPAD // GROUP
R = 100352
RSLICE = R // 16


NBUF = 4


def _sc_body(xtab, src2d, dst2d, wflat, zeros, out, table, vsrc, vdst, vw, rows,
             gsem, ssem, isem):
    cid = lax.axis_index("c")
    sid = lax.axis_index("s")
    wid = cid * 16 + sid

    @pl.when(sid == 0)
    def _zero():
        pltpu.sync_copy(zeros, table)

    plsc.subcore_barrier()

    iota = lax.broadcasted_iota(jnp.int32, (16,), 0)
    ci4 = iota & 3
    r4 = iota >> 2

    nck = jnp.where(cid == 0, NCK0, NCK1)
    cbase = jnp.where(cid == 0, sid * NCK0, 16 * NCK0 + sid * NCK1)

    def fire_idx(c, b):
        row0 = (cbase + c) * GROUPS_PER_CHUNK
        e0 = (cbase + c) * CHUNK
        pltpu.async_copy(src2d.at[pl.ds(row0, GROUPS_PER_CHUNK)], vsrc.at[b], isem)
        pltpu.async_copy(dst2d.at[pl.ds(row0, GROUPS_PER_CHUNK)], vdst.at[b], isem)
        pltpu.async_copy(wflat.at[pl.ds(e0, CHUNK)], vw.at[b], isem)

    def wait_idx(b):
        for _ in range(3):
            pltpu.make_async_copy(src2d.at[pl.ds(0, GROUPS_PER_CHUNK)],
                                  vsrc.at[b], isem).wait()

    def fire_gathers(b):
        for g in range(GROUPS_PER_CHUNK):
            pltpu.async_copy(xtab.at[vsrc.at[b, g]], rows.at[b, g], gsem)

    def wait_gathers(b):
        for g in range(GROUPS_PER_CHUNK):
            pltpu.make_async_copy(xtab.at[vsrc.at[b, g]], rows.at[b, g],
                                  gsem).wait()

    def fire_scatters(b):
        for g in range(GROUPS_PER_CHUNK):
            pltpu.async_copy(rows.at[b, g], table.at[vdst.at[b, g]], ssem,
                             add=True)

    def wait_scatters(b):
        for g in range(GROUPS_PER_CHUNK):
            pltpu.make_async_copy(rows.at[b, g], table.at[vdst.at[b, g]],
                                  ssem).wait()

    def mul(b):
        b_s = jnp.full((16,), b, jnp.int32)
        for g in range(GROUPS_PER_CHUNK):
            g_s = jnp.full((16,), g, jnp.int32)

            def _mb(i, carry):
                s = 32 * i
                for u in range(8):
                    ri = (s + 4 * u) + r4
                    wb = plsc.load_gather(vw, [b_s, g * GROUP + ri])
                    v = plsc.load_gather(rows, [b_s, g_s, ri, ci4])
                    plsc.store_scatter(rows, [b_s, g_s, ri, ci4], v * wb)
                return carry

            lax.fori_loop(0, GROUP // 32, _mb, 0)

    fire_idx(0, 0)
    wait_idx(0)
    fire_gathers(0)
    fire_idx(1, 1)

    def super_body(s, carry):
        for b in range(NBUF):
            c = NBUF * s + b
            b1 = (b + 1) % NBUF
            b2 = (b + 2) % NBUF

            @pl.when((c >= 2) & (c < nck + 2))
            def _():
                wait_scatters(b2)

            @pl.when(c + 2 < nck)
            def _():
                fire_idx(c + 2, b2)

            @pl.when(c < nck)
            def _():
                wait_gathers(b)
                mul(b)

            @pl.when(c + 1 < nck)
            def _():
                wait_idx(b1)
                fire_gathers(b1)

            @pl.when(c < nck)
            def _():
                fire_scatters(b)
        return carry

    lax.fori_loop(0, (nck + NBUF - 1) // NBUF, super_body, 0)

    plsc.subcore_barrier()
    pltpu.sync_copy(table.at[pl.ds(sid * RSLICE, RSLICE)],
                    out.at[cid, pl.ds(sid * RSLICE, RSLICE)])


def _sc_aggregate(xtab, src2d, dst2d, wflat, zeros):
    mesh = plsc.VectorSubcoreMesh(core_axis_name="c", subcore_axis_name="s")
    return pl.kernel(
        _sc_body,
        out_type=jax.ShapeDtypeStruct((2, R, 8), jnp.float32),
        mesh=mesh,
        compiler_params=pltpu.CompilerParams(needs_layout_passes=False,
                                             use_tc_tiling_on_sc=False),
        scratch_types=[
            pltpu.VMEM_SHARED((R, 8), jnp.float32),
            pltpu.VMEM((NBUF, GROUPS_PER_CHUNK, GROUP), jnp.int32),
            pltpu.VMEM((NBUF, GROUPS_PER_CHUNK, GROUP), jnp.int32),
            pltpu.VMEM((NBUF, CHUNK), jnp.float32),
            pltpu.VMEM((NBUF, GROUPS_PER_CHUNK, GROUP, 8), jnp.float32),
            pltpu.SemaphoreType.DMA,
            pltpu.SemaphoreType.DMA,
            pltpu.SemaphoreType.DMA,
        ],
    )(xtab, src2d, dst2d, wflat, zeros)


BLK = 4000


def _dot(a, b):
    return jnp.dot(a, b, preferred_element_type=jnp.float32)


def _tc_body(x_r, p_r, h0_r, c0_r, ggc_r,
             wir_r, whr_r, br_r, wiz_r, whz_r, bz_r,
             win_r, whn_r, bin_r, bhn_r,
             wii_r, whi_r, bi_r, wif_r, whf_r, bf_r,
             wig_r, whg_r, bg_r, wio_r, who_r, bo_r,
             lin_r, linb_r, out_r, ht_r, ct_r):
    p = p_r[...]
    aggx = p[0, :, 0:4] + p[1, :, 0:4]
    cnt = p[0, :, 4:5] + p[1, :, 4:5]
    agg = _dot(aggx, ggc_r[...]) / jnp.maximum(cnt, 1.0)
    xb = x_r[...]
    h0 = h0_r[...]
    r = jax.nn.sigmoid(_dot(agg, wir_r[...]) + _dot(xb, whr_r[...]) + br_r[...])
    z = jax.nn.sigmoid(_dot(agg, wiz_r[...]) + _dot(xb, whz_r[...]) + bz_r[...])
    nc = jnp.tanh(_dot(agg, win_r[...]) + bin_r[...]
                  + r * (_dot(xb, whn_r[...]) + bhn_r[...]))
    hc = (1.0 - z) * nc + z * xb
    i_t = jax.nn.sigmoid(_dot(hc, wii_r[...]) + _dot(h0, whi_r[...]) + bi_r[...])
    f_t = jax.nn.sigmoid(_dot(hc, wif_r[...]) + _dot(h0, whf_r[...]) + bf_r[...])
    g_t = jnp.tanh(_dot(hc, wig_r[...]) + _dot(h0, whg_r[...]) + bg_r[...])
    o_t = jax.nn.sigmoid(_dot(hc, wio_r[...]) + _dot(h0, who_r[...]) + bo_r[...])
    c_t = f_t * c0_r[...] + i_t * g_t
    h_t = o_t * jnp.tanh(c_t)
    ct_r[...] = c_t
    ht_r[...] = h_t
    out_r[...] = _dot(jax.nn.relu(h_t), lin_r[...]) + linb_r[...]


def _tc_dense(x, part, h_0, c_0, *weights):
    nblk = N // BLK
    full = lambda a: pl.BlockSpec(a.shape, lambda i: (0,) * a.ndim)
    return pl.pallas_call(
        _tc_body,
        grid=(nblk,),
        in_specs=[
            pl.BlockSpec((BLK, 4), lambda i: (i, 0)),
            pl.BlockSpec((2, BLK, 8), lambda i: (0, i, 0)),
            pl.BlockSpec((BLK, 32), lambda i: (i, 0)),
            pl.BlockSpec((BLK, 32), lambda i: (i, 0)),
        ] + [full(w) for w in weights],
        out_specs=[
            pl.BlockSpec((BLK, 1), lambda i: (i, 0)),
            pl.BlockSpec((BLK, 32), lambda i: (i, 0)),
            pl.BlockSpec((BLK, 32), lambda i: (i, 0)),
        ],
        out_shape=[
            jax.ShapeDtypeStruct((N, 1), jnp.float32),
            jax.ShapeDtypeStruct((N, 32), jnp.float32),
            jax.ShapeDtypeStruct((N, 32), jnp.float32),
        ],
    )(x, part, h_0, c_0, *weights)


def kernel(x, edge_index, edge_weight, h_0, c_0, ggc_w, gru_wi, gru_wh,
           gru_bi, gru_bh, lstm_wi, lstm_wh, lstm_bi, lstm_bh, lin_w, lin_b):
    pad = EPAD - E
    src_p = jnp.concatenate([edge_index[0], jnp.zeros((pad,), jnp.int32)])
    pad_dst = N + (jnp.arange(pad, dtype=jnp.int32) & 127)
    dst_p = jnp.concatenate([edge_index[1], pad_dst])
    w_p = jnp.concatenate([edge_weight, jnp.zeros((pad,), jnp.float32)])
    src2d = src_p.reshape(ROWS2D, GROUP)
    dst2d = dst_p.reshape(ROWS2D, GROUP)
    xtab = jnp.concatenate(
        [x, jnp.ones((N, 1), jnp.float32), jnp.zeros((N, 3), jnp.float32)],
        axis=1)
    zeros = jnp.zeros((R, 8), jnp.float32)

    part = _sc_aggregate(xtab, src2d, dst2d, w_p, zeros)

    gwi, gwh = gru_wi.T, gru_wh.T
    lwi, lwh = lstm_wi.T, lstm_wh.T
    gb = gru_bi + gru_bh
    lb = lstm_bi + lstm_bh
    weights = [
        ggc_w,
        gwi[:, 0:4], gwh[:, 0:4], gb[0:4].reshape(1, 4),
        gwi[:, 4:8], gwh[:, 4:8], gb[4:8].reshape(1, 4),
        gwi[:, 8:12], gwh[:, 8:12],
        gru_bi[8:12].reshape(1, 4), gru_bh[8:12].reshape(1, 4),
        lwi[:, 0:32], lwh[:, 0:32], lb[0:32].reshape(1, 32),
        lwi[:, 32:64], lwh[:, 32:64], lb[32:64].reshape(1, 32),
        lwi[:, 64:96], lwh[:, 64:96], lb[64:96].reshape(1, 32),
        lwi[:, 96:128], lwh[:, 96:128], lb[96:128].reshape(1, 32),
        lin_w.T, lin_b.reshape(1, 1),
    ]
    out, h_t, c_t = _tc_dense(x, part, h_0, c_0, *weights)
    return (out, h_t, c_t)
